# baseline (speedup 1.0000x reference)
import functools

import jax
import jax.numpy as jnp
from jax import lax
from jax.experimental import pallas as pl
from jax.experimental.pallas import tpu as pltpu
from jax.experimental.pallas import tpu_sc as plsc

N, H, E, K, I = 2048, 2048, 8, 2, 1408
TM = 128
NBR = (N * K + E * (TM - 1) + TM - 1) // TM
PR = NBR * TM
NBB = ((NBR + 7) // 8) * 8
SM = 256
NEG = -1e30

INTERP = False


def _router_body(x_ref, gw_ref, wts_ref, dests_ref, bexp_ref):
    x = x_ref[...]
    gw = gw_ref[...]
    logits = lax.dot_general(x, gw, (((1,), (1,)), ((), ())),
                             preferred_element_type=jnp.float32)
    lanes = lax.broadcasted_iota(jnp.int32, (N, E), 1)
    m1 = jnp.max(logits, axis=1, keepdims=True)
    i1 = jnp.min(jnp.where(logits == m1, lanes, E), axis=1, keepdims=True)
    masked = jnp.where(lanes == i1, NEG, logits)
    m2 = jnp.max(masked, axis=1, keepdims=True)
    i2 = jnp.min(jnp.where(masked == m2, lanes, E), axis=1, keepdims=True)
    w0 = 1.0 / (1.0 + jnp.exp(m2 - m1))
    w1_ = 1.0 - w0
    m0mask = (lanes == i1).astype(jnp.float32)
    m1mask = (lanes == i2).astype(jnp.float32)
    rr = lax.broadcasted_iota(jnp.int32, (N, N), 0)
    cc = lax.broadcasted_iota(jnp.int32, (N, N), 1)
    lt = (rr >= cc).astype(jnp.float32)
    c0 = lax.dot_general(lt, m0mask, (((1,), (0,)), ((), ())),
                         preferred_element_type=jnp.float32)
    c1 = lax.dot_general(lt, m1mask, (((1,), (0,)), ((), ())),
                         preferred_element_type=jnp.float32)
    tot0 = c0[N - 1:N, :]
    tot1 = c1[N - 1:N, :]
    cnt = tot0 + tot1
    pc = jnp.ceil(cnt / TM) * TM
    er = lax.broadcasted_iota(jnp.int32, (E, E), 0)
    ec = lax.broadcasted_iota(jnp.int32, (E, E), 1)
    sut = (er < ec).astype(jnp.float32)
    base = lax.dot_general(pc, sut, (((1,), (0,)), ((), ())),
                           preferred_element_type=jnp.float32)
    d0 = jnp.sum(m0mask * (base + c0 - 1.0), axis=1, keepdims=True)
    d1 = jnp.sum(m1mask * (base + tot0 + c1 - 1.0), axis=1, keepdims=True)
    dests_ref[...] = jnp.concatenate(
        [d0.astype(jnp.int32), d1.astype(jnp.int32),
         jnp.zeros((N, 6), jnp.int32)], axis=1)
    wts_ref[...] = jnp.concatenate(
        [w0, w1_, jnp.zeros((N, 6), jnp.float32)], axis=1)
    incl = base + pc
    bs = lax.broadcasted_iota(jnp.int32, (NBB, 1), 0).astype(jnp.float32) * TM
    be = jnp.sum((bs >= incl).astype(jnp.float32), axis=1, keepdims=True)
    be = jnp.minimum(be, float(E - 1))
    nact = jnp.sum(pc, axis=1, keepdims=True) / TM
    bexp_ref[...] = jnp.concatenate(
        [be.astype(jnp.int32),
         jnp.broadcast_to(nact.astype(jnp.int32), (NBB, 1)),
         jnp.zeros((NBB, 6), jnp.int32)], axis=1)


def _router(x, gw):
    return pl.pallas_call(
        _router_body,
        grid=(1,),
        in_specs=[pl.BlockSpec((N, H), lambda i: (0, 0)),
                  pl.BlockSpec((E, H), lambda i: (0, 0))],
        out_specs=[pl.BlockSpec((N, 8), lambda i: (0, 0)),
                   pl.BlockSpec((N, 8), lambda i: (0, 0)),
                   pl.BlockSpec((NBB, 8), lambda i: (0, 0))],
        out_shape=[jax.ShapeDtypeStruct((N, 8), jnp.float32),
                   jax.ShapeDtypeStruct((N, 8), jnp.int32),
                   jax.ShapeDtypeStruct((NBB, 8), jnp.int32)],
        interpret=INTERP,
    )(x, gw)


def _sc_scatter_body(x_hbm, drs_hbm, xg_hbm, xbuf, idxb, sem):
    w = lax.axis_index("c") * 16 + lax.axis_index("s")
    pltpu.sync_copy(drs_hbm.at[pl.ds(w * 4, 4)], idxb)
    for i in range(4):
        n0 = (w % 16) * 128 + i * 32
        pltpu.sync_copy(x_hbm.at[pl.ds(n0, 32)], xbuf)
        pltpu.async_copy(xbuf, xg_hbm.at[idxb.at[i]], sem).wait()


def _sc_scatter(x, drs):
    mesh = plsc.VectorSubcoreMesh(core_axis_name="c", subcore_axis_name="s")
    return pl.kernel(
        _sc_scatter_body,
        out_type=jax.ShapeDtypeStruct((PR, H), jnp.float32),
        mesh=mesh,
        scratch_types=[pltpu.VMEM((32, H), jnp.float32),
                       pltpu.VMEM((4, 32), jnp.int32),
                       pltpu.SemaphoreType.DMA],
    )(x, drs)


def _gemm1_body(bexp_ref, na_ref, xg_ref, wg_ref, wu_ref, act_ref):
    @pl.when(pl.program_id(0) < na_ref[0])
    def _():
        xb = xg_ref[...].astype(jnp.bfloat16)
        g = jnp.dot(xb, wg_ref[0].astype(jnp.bfloat16),
                    preferred_element_type=jnp.float32)
        u = jnp.dot(xb, wu_ref[0].astype(jnp.bfloat16),
                    preferred_element_type=jnp.float32)
        act_ref[...] = (g / (1.0 + jnp.exp(-g)) * u).astype(jnp.bfloat16)


def _gemm1(bexp, nact, xg, w1r):
    grid_spec = pltpu.PrefetchScalarGridSpec(
        num_scalar_prefetch=2,
        grid=(NBR,),
        in_specs=[
            pl.BlockSpec((TM, H), lambda nb, be, na: (jnp.minimum(nb, na[0]), 0)),
            pl.BlockSpec((1, H, I), lambda nb, be, na: (be[nb], 0, 0)),
            pl.BlockSpec((1, H, I), lambda nb, be, na: (be[nb], 0, 1)),
        ],
        out_specs=pl.BlockSpec(
            (TM, I), lambda nb, be, na: (jnp.minimum(nb, na[0]), 0)),
    )
    return pl.pallas_call(
        _gemm1_body,
        grid_spec=grid_spec,
        out_shape=jax.ShapeDtypeStruct((PR, I), jnp.bfloat16),
        compiler_params=pltpu.CompilerParams(
            dimension_semantics=("arbitrary",),
            vmem_limit_bytes=60 * 1024 * 1024),
        interpret=INTERP,
    )(bexp, nact, xg, w1r, w1r)


def _gemm2_body(bexp_ref, na_ref, act_ref, w2_ref, y_ref):
    @pl.when(pl.program_id(0) < na_ref[0])
    def _():
        y_ref[...] = jnp.dot(act_ref[...], w2_ref[0].astype(jnp.bfloat16),
                             preferred_element_type=jnp.float32)


def _gemm2(bexp, nact, act_r, w2r):
    grid_spec = pltpu.PrefetchScalarGridSpec(
        num_scalar_prefetch=2,
        grid=(NBR,),
        in_specs=[
            pl.BlockSpec((TM, I), lambda nb, be, na: (jnp.minimum(nb, na[0]), 0)),
            pl.BlockSpec((1, I, H), lambda nb, be, na: (be[nb], 0, 0)),
        ],
        out_specs=pl.BlockSpec(
            (TM, H), lambda nb, be, na: (jnp.minimum(nb, na[0]), 0)),
    )
    return pl.pallas_call(
        _gemm2_body,
        grid_spec=grid_spec,
        out_shape=jax.ShapeDtypeStruct((PR, H), jnp.float32),
        compiler_params=pltpu.CompilerParams(
            dimension_semantics=("arbitrary",)),
        interpret=INTERP,
    )(bexp, nact, act_r, w2r)


def _sc_gather_body(y_hbm, dg_hbm, yg0_hbm, yg1_hbm, rows, idxb, sem):
    w = lax.axis_index("c") * 16 + lax.axis_index("s")
    n0 = w * 64
    for k in range(2):
        out = yg0_hbm if k == 0 else yg1_hbm
        pltpu.sync_copy(dg_hbm.at[k * 32 + w], idxb)
        for i in range(2):
            pltpu.async_copy(y_hbm.at[idxb.at[pl.ds(i * 32, 32)]],
                             rows, sem).wait()
            pltpu.sync_copy(rows, out.at[pl.ds(n0 + i * 32, 32)])


def _sc_gather(y, dg):
    mesh = plsc.VectorSubcoreMesh(core_axis_name="c", subcore_axis_name="s")
    return pl.kernel(
        _sc_gather_body,
        out_type=[jax.ShapeDtypeStruct((N, H), jnp.float32),
                  jax.ShapeDtypeStruct((N, H), jnp.float32)],
        mesh=mesh,
        scratch_types=[pltpu.VMEM((32, H), jnp.float32),
                       pltpu.VMEM((64,), jnp.int32),
                       pltpu.SemaphoreType.DMA],
    )(y, dg)


def _sgemm1_body(x_ref, sw1_ref, sw2_ref, act_ref):
    xb = x_ref[...].astype(jnp.bfloat16)
    g = lax.dot_general(xb, sw1_ref[...].astype(jnp.bfloat16),
                        (((1,), (1,)), ((), ())),
                        preferred_element_type=jnp.float32)
    u = lax.dot_general(xb, sw2_ref[...].astype(jnp.bfloat16),
                        (((1,), (1,)), ((), ())),
                        preferred_element_type=jnp.float32)
    act_ref[...] = (g / (1.0 + jnp.exp(-g)) * u).astype(jnp.bfloat16)


def _sgemm1(x, sw1, sw2):
    return pl.pallas_call(
        _sgemm1_body,
        grid=(N // SM,),
        in_specs=[pl.BlockSpec((SM, H), lambda nb: (nb, 0)),
                  pl.BlockSpec((I, H), lambda nb: (0, 0)),
                  pl.BlockSpec((I, H), lambda nb: (0, 0))],
        out_specs=pl.BlockSpec((SM, I), lambda nb: (nb, 0)),
        out_shape=jax.ShapeDtypeStruct((N, I), jnp.bfloat16),
        interpret=INTERP,
    )(x, sw1, sw2)


def _sgemm2_body(act_ref, sw3_ref, yg0_ref, yg1_ref, wts_ref, out_ref):
    s = lax.dot_general(act_ref[...], sw3_ref[...].astype(jnp.bfloat16),
                        (((1,), (1,)), ((), ())),
                        preferred_element_type=jnp.float32)
    w0 = wts_ref[:, 0:1]
    w1_ = wts_ref[:, 1:2]
    out_ref[...] = s + w0 * yg0_ref[...] + w1_ * yg1_ref[...]


def _sgemm2(act_s, sw3, yg0, yg1, wts):
    return pl.pallas_call(
        _sgemm2_body,
        grid=(N // SM,),
        in_specs=[pl.BlockSpec((SM, I), lambda nb: (nb, 0)),
                  pl.BlockSpec((H, I), lambda nb: (0, 0)),
                  pl.BlockSpec((SM, H), lambda nb: (nb, 0)),
                  pl.BlockSpec((SM, H), lambda nb: (nb, 0)),
                  pl.BlockSpec((SM, 8), lambda nb: (nb, 0))],
        out_specs=pl.BlockSpec((SM, H), lambda nb: (nb, 0)),
        out_shape=jax.ShapeDtypeStruct((N, H), jnp.float32),
        interpret=INTERP,
    )(act_s, sw3, yg0, yg1, wts)


def kernel(hidden_states, gate_weight, w1, w2, sw1, sw2, sw3):
    x = hidden_states.reshape(N, H)
    wts, dests, bexp32 = _router(x, gate_weight)
    bexp = bexp32[:NBR, 0]
    nact = bexp32[0:1, 1]
    d0 = dests[:, 0]
    d1 = dests[:, 1]
    dflat = jnp.concatenate([d0, d1])
    drs = dflat.reshape(128, 32)
    dg = dflat.reshape(64, 64)
    act_s = _sgemm1(x, sw1, sw2)
    xg = _sc_scatter(x, drs)
    w1r = w1.reshape(E, H, 2 * I)
    act_r = _gemm1(bexp, nact, xg, w1r)
    w2r = w2.reshape(E, I, H)
    y = _gemm2(bexp, nact, act_r, w2r)
    yg0, yg1 = _sc_gather(y, dg)
    out = _sgemm2(act_s, sw3, yg0, yg1, wts)
    return out.reshape(1, N, H)


# speedup vs baseline: 1.0212x; 1.0212x over previous
import jax
import jax.numpy as jnp
from jax import lax
from jax.experimental import pallas as pl
from jax.experimental.pallas import tpu as pltpu
from jax.experimental.pallas import tpu_sc as plsc

N, H, E, K, I = 2048, 2048, 8, 2, 1408
TM = 256
NBR = (N * K + E * (TM - 1) + TM - 1) // TM
PR = NBR * TM
NBB = ((NBR + 7) // 8) * 8
SM = 256
NEG = -1e30


def _router_body(x_ref, gw_ref, wts_ref, dests_ref, bexp_ref):
    x = x_ref[...]
    gw = gw_ref[...]
    logits = lax.dot_general(x, gw, (((1,), (1,)), ((), ())),
                             preferred_element_type=jnp.float32)
    lanes = lax.broadcasted_iota(jnp.int32, (N, E), 1)
    m1 = jnp.max(logits, axis=1, keepdims=True)
    i1 = jnp.min(jnp.where(logits == m1, lanes, E), axis=1, keepdims=True)
    masked = jnp.where(lanes == i1, NEG, logits)
    m2 = jnp.max(masked, axis=1, keepdims=True)
    i2 = jnp.min(jnp.where(masked == m2, lanes, E), axis=1, keepdims=True)
    w0 = 1.0 / (1.0 + jnp.exp(m2 - m1))
    w1_ = 1.0 - w0
    m0mask = (lanes == i1).astype(jnp.float32)
    m1mask = (lanes == i2).astype(jnp.float32)
    rr = lax.broadcasted_iota(jnp.int32, (N, N), 0)
    cc = lax.broadcasted_iota(jnp.int32, (N, N), 1)
    lt = (rr >= cc).astype(jnp.float32)
    c0 = lax.dot_general(lt, m0mask, (((1,), (0,)), ((), ())),
                         preferred_element_type=jnp.float32)
    c1 = lax.dot_general(lt, m1mask, (((1,), (0,)), ((), ())),
                         preferred_element_type=jnp.float32)
    tot0 = c0[N - 1:N, :]
    tot1 = c1[N - 1:N, :]
    cnt = tot0 + tot1
    pc = jnp.ceil(cnt / TM) * TM
    er = lax.broadcasted_iota(jnp.int32, (E, E), 0)
    ec = lax.broadcasted_iota(jnp.int32, (E, E), 1)
    sut = (er < ec).astype(jnp.float32)
    base = lax.dot_general(pc, sut, (((1,), (0,)), ((), ())),
                           preferred_element_type=jnp.float32)
    d0 = jnp.sum(m0mask * (base + c0 - 1.0), axis=1, keepdims=True)
    d1 = jnp.sum(m1mask * (base + tot0 + c1 - 1.0), axis=1, keepdims=True)
    dests_ref[...] = jnp.concatenate(
        [d0.astype(jnp.int32), d1.astype(jnp.int32),
         jnp.zeros((N, 6), jnp.int32)], axis=1)
    wts_ref[...] = jnp.concatenate(
        [w0, w1_, jnp.zeros((N, 6), jnp.float32)], axis=1)
    incl = base + pc
    bs = lax.broadcasted_iota(jnp.int32, (NBB, 1), 0).astype(jnp.float32) * TM
    be = jnp.sum((bs >= incl).astype(jnp.float32), axis=1, keepdims=True)
    be = jnp.minimum(be, float(E - 1))
    nact = jnp.sum(pc, axis=1, keepdims=True) / TM
    bexp_ref[...] = jnp.concatenate(
        [be.astype(jnp.int32),
         jnp.broadcast_to(nact.astype(jnp.int32), (NBB, 1)),
         jnp.zeros((NBB, 6), jnp.int32)], axis=1)


def _router(x, gw):
    return pl.pallas_call(
        _router_body,
        grid=(1,),
        in_specs=[pl.BlockSpec((N, H), lambda i: (0, 0)),
                  pl.BlockSpec((E, H), lambda i: (0, 0))],
        out_specs=[pl.BlockSpec((N, 8), lambda i: (0, 0)),
                   pl.BlockSpec((N, 8), lambda i: (0, 0)),
                   pl.BlockSpec((NBB, 8), lambda i: (0, 0))],
        out_shape=[jax.ShapeDtypeStruct((N, 8), jnp.float32),
                   jax.ShapeDtypeStruct((N, 8), jnp.int32),
                   jax.ShapeDtypeStruct((NBB, 8), jnp.int32)],
    )(x, gw)


def _sc_scatter_body(x_hbm, drs_hbm, xg_hbm, xbuf, idxb, sem):
    w = lax.axis_index("c") * 16 + lax.axis_index("s")
    pltpu.sync_copy(drs_hbm.at[pl.ds(w * 4, 4)], idxb)
    for i in range(4):
        n0 = (w % 16) * 128 + i * 32
        pltpu.sync_copy(x_hbm.at[pl.ds(n0, 32)], xbuf)
        pltpu.async_copy(xbuf, xg_hbm.at[idxb.at[i]], sem).wait()


def _sc_scatter(x, drs):
    mesh = plsc.VectorSubcoreMesh(core_axis_name="c", subcore_axis_name="s")
    return pl.kernel(
        _sc_scatter_body,
        out_type=jax.ShapeDtypeStruct((PR, H), jnp.float32),
        mesh=mesh,
        scratch_types=[pltpu.VMEM((32, H), jnp.float32),
                       pltpu.VMEM((4, 32), jnp.int32),
                       pltpu.SemaphoreType.DMA],
    )(x, drs)


def _gemm1_body(bexp_ref, na_ref, xg_ref, wg_ref, wu_ref, act_ref):
    @pl.when(pl.program_id(0) < na_ref[0])
    def _():
        xb = xg_ref[...].astype(jnp.bfloat16)
        g = jnp.dot(xb, wg_ref[0].astype(jnp.bfloat16),
                    preferred_element_type=jnp.float32)
        u = jnp.dot(xb, wu_ref[0].astype(jnp.bfloat16),
                    preferred_element_type=jnp.float32)
        act_ref[...] = (g / (1.0 + jnp.exp(-g)) * u).astype(jnp.bfloat16)


def _gemm1(bexp, nact, xg, w1r):
    grid_spec = pltpu.PrefetchScalarGridSpec(
        num_scalar_prefetch=2,
        grid=(NBR,),
        in_specs=[
            pl.BlockSpec((TM, H), lambda nb, be, na: (jnp.minimum(nb, na[0]), 0)),
            pl.BlockSpec((1, H, I), lambda nb, be, na: (be[nb], 0, 0)),
            pl.BlockSpec((1, H, I), lambda nb, be, na: (be[nb], 0, 1)),
        ],
        out_specs=pl.BlockSpec(
            (TM, I), lambda nb, be, na: (jnp.minimum(nb, na[0]), 0)),
    )
    return pl.pallas_call(
        _gemm1_body,
        grid_spec=grid_spec,
        out_shape=jax.ShapeDtypeStruct((PR, I), jnp.bfloat16),
        compiler_params=pltpu.CompilerParams(
            dimension_semantics=("arbitrary",),
            vmem_limit_bytes=60 * 1024 * 1024),
    )(bexp, nact, xg, w1r, w1r)


def _gemm2_body(bexp_ref, na_ref, act_ref, w2_ref, y_ref):
    @pl.when(pl.program_id(0) < na_ref[0])
    def _():
        y_ref[...] = jnp.dot(act_ref[...], w2_ref[0].astype(jnp.bfloat16),
                             preferred_element_type=jnp.float32)


def _gemm2(bexp, nact, act_r, w2r):
    grid_spec = pltpu.PrefetchScalarGridSpec(
        num_scalar_prefetch=2,
        grid=(NBR,),
        in_specs=[
            pl.BlockSpec((TM, I), lambda nb, be, na: (jnp.minimum(nb, na[0]), 0)),
            pl.BlockSpec((1, I, H), lambda nb, be, na: (be[nb], 0, 0)),
        ],
        out_specs=pl.BlockSpec(
            (TM, H), lambda nb, be, na: (jnp.minimum(nb, na[0]), 0)),
    )
    return pl.pallas_call(
        _gemm2_body,
        grid_spec=grid_spec,
        out_shape=jax.ShapeDtypeStruct((PR, H), jnp.float32),
        compiler_params=pltpu.CompilerParams(
            dimension_semantics=("arbitrary",)),
    )(bexp, nact, act_r, w2r)


def _sc_gather_body(y_hbm, dg_hbm, yg0_hbm, yg1_hbm, rows, idxb, sem):
    w = lax.axis_index("c") * 16 + lax.axis_index("s")
    n0 = w * 64
    for k in range(2):
        out = yg0_hbm if k == 0 else yg1_hbm
        pltpu.sync_copy(dg_hbm.at[k * 32 + w], idxb)
        for i in range(2):
            pltpu.async_copy(y_hbm.at[idxb.at[pl.ds(i * 32, 32)]],
                             rows, sem).wait()
            pltpu.sync_copy(rows, out.at[pl.ds(n0 + i * 32, 32)])


def _sc_gather(y, dg):
    mesh = plsc.VectorSubcoreMesh(core_axis_name="c", subcore_axis_name="s")
    return pl.kernel(
        _sc_gather_body,
        out_type=[jax.ShapeDtypeStruct((N, H), jnp.float32),
                  jax.ShapeDtypeStruct((N, H), jnp.float32)],
        mesh=mesh,
        scratch_types=[pltpu.VMEM((32, H), jnp.float32),
                       pltpu.VMEM((64,), jnp.int32),
                       pltpu.SemaphoreType.DMA],
    )(y, dg)


def _sgemm1_body(x_ref, sw1_ref, sw2_ref, act_ref):
    xb = x_ref[...].astype(jnp.bfloat16)
    g = lax.dot_general(xb, sw1_ref[...].astype(jnp.bfloat16),
                        (((1,), (1,)), ((), ())),
                        preferred_element_type=jnp.float32)
    u = lax.dot_general(xb, sw2_ref[...].astype(jnp.bfloat16),
                        (((1,), (1,)), ((), ())),
                        preferred_element_type=jnp.float32)
    act_ref[...] = (g / (1.0 + jnp.exp(-g)) * u).astype(jnp.bfloat16)


def _sgemm1(x, sw1, sw2):
    return pl.pallas_call(
        _sgemm1_body,
        grid=(N // SM,),
        in_specs=[pl.BlockSpec((SM, H), lambda nb: (nb, 0)),
                  pl.BlockSpec((I, H), lambda nb: (0, 0)),
                  pl.BlockSpec((I, H), lambda nb: (0, 0))],
        out_specs=pl.BlockSpec((SM, I), lambda nb: (nb, 0)),
        out_shape=jax.ShapeDtypeStruct((N, I), jnp.bfloat16),
    )(x, sw1, sw2)


def _sgemm2_body(act_ref, sw3_ref, yg0_ref, yg1_ref, wts_ref, out_ref):
    s = lax.dot_general(act_ref[...], sw3_ref[...].astype(jnp.bfloat16),
                        (((1,), (1,)), ((), ())),
                        preferred_element_type=jnp.float32)
    w0 = wts_ref[:, 0:1]
    w1_ = wts_ref[:, 1:2]
    out_ref[...] = s + w0 * yg0_ref[...] + w1_ * yg1_ref[...]


def _sgemm2(act_s, sw3, yg0, yg1, wts):
    return pl.pallas_call(
        _sgemm2_body,
        grid=(N // SM,),
        in_specs=[pl.BlockSpec((SM, I), lambda nb: (nb, 0)),
                  pl.BlockSpec((H, I), lambda nb: (0, 0)),
                  pl.BlockSpec((SM, H), lambda nb: (nb, 0)),
                  pl.BlockSpec((SM, H), lambda nb: (nb, 0)),
                  pl.BlockSpec((SM, 8), lambda nb: (nb, 0))],
        out_specs=pl.BlockSpec((SM, H), lambda nb: (nb, 0)),
        out_shape=jax.ShapeDtypeStruct((N, H), jnp.float32),
    )(act_s, sw3, yg0, yg1, wts)


def kernel(hidden_states, gate_weight, w1, w2, sw1, sw2, sw3):
    x = hidden_states.reshape(N, H)
    wts, dests, bexp32 = _router(x, gate_weight)
    bexp = bexp32[:NBR, 0]
    nact = bexp32[0:1, 1]
    d0 = dests[:, 0]
    d1 = dests[:, 1]
    dflat = jnp.concatenate([d0, d1])
    drs = dflat.reshape(128, 32)
    dg = dflat.reshape(64, 64)
    act_s = _sgemm1(x, sw1, sw2)
    xg = _sc_scatter(x, drs)
    w1r = w1.reshape(E, H, 2 * I)
    act_r = _gemm1(bexp, nact, xg, w1r)
    w2r = w2.reshape(E, I, H)
    y = _gemm2(bexp, nact, act_r, w2r)
    yg0, yg1 = _sc_gather(y, dg)
    out = _sgemm2(act_s, sw3, yg0, yg1, wts)
    return out.reshape(1, N, H)
